# baseline (speedup 1.0000x reference)
import functools

import jax
import jax.numpy as jnp
from jax import lax
from jax.experimental import pallas as pl
from jax.experimental.pallas import tpu as pltpu
from jax.experimental.pallas import tpu_sc as plsc

N, E, D = 10000, 320000, 128
L = 16
NC, NS = 2, 16
NW = NC * NS
EH = E // 2
EW = EH // NW
C = 40
CHUNKS = EW // C
PAIRS = CHUNKS // 2

_HI = lax.Precision.HIGHEST


def _dot(a, b):
    return lax.dot_general(a, b, (((1,), (0,)), ((), ())),
                           precision=_HI, preferred_element_type=jnp.float32)


def _silu(x):
    return x * jax.nn.sigmoid(x)


def _pack_bf16_pairs(x):
    lo = lax.bitcast_convert_type(x[:, :D // 2].astype(jnp.bfloat16),
                                  jnp.uint16).astype(jnp.int32)
    hi = lax.bitcast_convert_type(x[:, D // 2:].astype(jnp.bfloat16),
                                  jnp.uint16).astype(jnp.int32)
    return lo | (hi << 16)


_BE = 1600


def _fused_proj_body(nodes_ref, w1a_ref, b1_ref, ef_ref, w1b_ref,
                     p_ref, efp_ref):
    @pl.when(pl.program_id(0) == 0)
    def _():
        p_ref[...] = _dot(nodes_ref[...], w1a_ref[...]) + b1_ref[...]

    efp_ref[...] = _pack_bf16_pairs(_dot(ef_ref[...], w1b_ref[...]))


def _node_and_edge_proj(nodes, w1a, b1, ef, w1b):
    grid = EH // _BE + 1
    emap = lambda i: (jnp.maximum(i - 1, 0), 0)
    eomap = lambda i: (jnp.maximum(i - 1, 0), 0)
    whole = lambda i: (0, 0)
    return pl.pallas_call(
        _fused_proj_body,
        grid=(grid,),
        in_specs=[
            pl.BlockSpec((N, D), whole),
            pl.BlockSpec((D, D), whole),
            pl.BlockSpec((1, D), whole),
            pl.BlockSpec((_BE, D), emap),
            pl.BlockSpec((D, D), whole),
        ],
        out_specs=[
            pl.BlockSpec((N, D), whole),
            pl.BlockSpec((_BE, D // 2), eomap),
        ],
        out_shape=[
            jax.ShapeDtypeStruct((N, D), jnp.float32),
            jax.ShapeDtypeStruct((EH, D // 2), jnp.int32),
        ],
    )(nodes, w1a, b1.reshape(1, D), ef, w1b)


def _edge_proj_body(ef_ref, w_ref, out_ref):
    out_ref[...] = _pack_bf16_pairs(_dot(ef_ref[...], w_ref[...]))


def _edge_proj(ef, w):
    grid = EH // _BE
    return pl.pallas_call(
        _edge_proj_body,
        grid=(grid,),
        in_specs=[
            pl.BlockSpec((_BE, D), lambda i: (i + EH // _BE, 0)),
            pl.BlockSpec((D, D), lambda i: (0, 0)),
        ],
        out_specs=pl.BlockSpec((_BE, D // 2), lambda i: (i, 0)),
        out_shape=jax.ShapeDtypeStruct((EH, D // 2), jnp.int32),
    )(ef, w)


_ZR = 640
_ZR_LAST = N - 15 * _ZR


def _sc_body(edge0, p_hbm, efp_hbm, src_hbm, tgt_hbm, init_hbm, out_hbm,
             srcall, tidx0, tidx1, g0, g1, e0, e1, m0, m1,
             acc, tsems, gsems, esems, ssems):
    c = lax.axis_index("c")
    s = lax.axis_index("s")
    wid = s * NC + c
    base_e = edge0 + wid * EW
    base_f = wid * EW
    tidx = (tidx0, tidx1)
    grows = (g0, g1)
    erows = (e0, e1)
    mrows = (m0, m1)

    @pl.when(s < NS - 1)
    def _():
        pltpu.sync_copy(init_hbm.at[c, pl.ds(s * _ZR, _ZR)],
                        acc.at[pl.ds(s * _ZR, _ZR)])

    @pl.when(s == NS - 1)
    def _():
        pltpu.sync_copy(init_hbm.at[c, pl.ds(15 * _ZR, _ZR_LAST)],
                        acc.at[pl.ds(15 * _ZR, _ZR_LAST)])

    pltpu.sync_copy(src_hbm.at[pl.ds(base_e, EW)], srcall)
    plsc.subcore_barrier()

    def fetch(b, t):
        pltpu.async_copy(tgt_hbm.at[pl.ds(base_e + t * C, C)], tidx[b],
                         tsems.at[b])
        pltpu.async_copy(p_hbm.at[srcall.at[pl.ds(t * C, C)]], grows[b],
                         gsems.at[b])
        pltpu.async_copy(efp_hbm.at[pl.ds(base_f + t * C, C)], erows[b],
                         esems.at[b])

    def wait_fetch(b, t):
        pltpu.make_async_copy(tgt_hbm.at[pl.ds(base_e + t * C, C)], tidx[b],
                              tsems.at[b]).wait()
        pltpu.make_async_copy(p_hbm.at[srcall.at[pl.ds(t * C, C)]], grows[b],
                              gsems.at[b]).wait()
        pltpu.make_async_copy(efp_hbm.at[pl.ds(base_f + t * C, C)], erows[b],
                              esems.at[b]).wait()

    def compute(b):
        gb = grows[b]
        eb = erows[b]
        mb = mrows[b]

        _f32 = lambda v: lax.bitcast_convert_type(v, jnp.float32)

        def row_body(i):
            for a in range(D // (2 * L)):
                ew = eb[i, pl.ds(a * L, L)]
                x = gb[i, pl.ds(a * L, L)] + _f32(ew << 16)
                mb[i, pl.ds(a * L, L)] = x / (1.0 + jnp.exp(-x))
                x2 = (gb[i, pl.ds(D // 2 + a * L, L)]
                      + _f32(ew & jnp.int32(-65536)))
                mb[i, pl.ds(D // 2 + a * L, L)] = x2 / (1.0 + jnp.exp(-x2))

        plsc.parallel_loop(0, C, unroll=2)(row_body)

    def process(b, t):
        wait_fetch(b, t)
        compute(b)
        pltpu.async_copy(mrows[b], acc.at[tidx[b]], ssems.at[b], add=True)
        pltpu.make_async_copy(mrows[b], acc.at[tidx[b]], ssems.at[b]).wait()

        @pl.when(t + 2 < CHUNKS)
        def _():
            fetch(b, t + 2)

    fetch(0, 0)
    fetch(1, 1)

    def pair_body(it, carry):
        process(0, it * 2)
        process(1, it * 2 + 1)
        return carry

    lax.fori_loop(0, PAIRS, pair_body, 0)
    process(0, CHUNKS - 1)

    plsc.subcore_barrier()

    @pl.when(s < NS - 1)
    def _():
        pltpu.sync_copy(acc.at[pl.ds(s * _ZR, _ZR)],
                        out_hbm.at[c, pl.ds(s * _ZR, _ZR)])

    @pl.when(s == NS - 1)
    def _():
        pltpu.sync_copy(acc.at[pl.ds(15 * _ZR, _ZR_LAST)],
                        out_hbm.at[c, pl.ds(15 * _ZR, _ZR_LAST)])


def _make_sc_aggregate(edge0):
    @functools.partial(
        pl.kernel,
        out_type=jax.ShapeDtypeStruct((NC, N, D), jnp.float32),
        mesh=plsc.VectorSubcoreMesh(core_axis_name="c", subcore_axis_name="s"),
        scratch_types=[
            pltpu.VMEM((EW,), jnp.int32),
            pltpu.VMEM((C,), jnp.int32),
            pltpu.VMEM((C,), jnp.int32),
            pltpu.VMEM((C, D), jnp.float32),
            pltpu.VMEM((C, D), jnp.float32),
            pltpu.VMEM((C, D // 2), jnp.int32),
            pltpu.VMEM((C, D // 2), jnp.int32),
            pltpu.VMEM((C, D), jnp.float32),
            pltpu.VMEM((C, D), jnp.float32),
            pltpu.VMEM_SHARED((N, D), jnp.float32),
            pltpu.SemaphoreType.DMA((2,)),
            pltpu.SemaphoreType.DMA((2,)),
            pltpu.SemaphoreType.DMA((2,)),
            pltpu.SemaphoreType.DMA((2,)),
        ],
    )
    def sc_aggregate(*args):
        _sc_body(edge0, *args)

    return sc_aggregate


_sc_half0 = _make_sc_aggregate(0)
_sc_half1 = _make_sc_aggregate(EH)


_BN = 2000


def _update_body(nodes_ref, part_ref, w2a_ref, w2b_ref, b2_ref,
                 w3_ref, b3_ref, out_ref):
    nodes = nodes_ref[...]
    agg = part_ref[0] + part_ref[1]
    u = _silu(_dot(nodes, w2a_ref[...]) + _dot(agg, w2b_ref[...]) + b2_ref[...])
    out_ref[...] = nodes + _dot(u, w3_ref[...]) + b3_ref[...]


def _node_update(nodes, partials, w2a, w2b, b2, w3, b3):
    grid = N // _BN
    blk = lambda i: (i, 0)
    whole = lambda i: (0, 0)
    return pl.pallas_call(
        _update_body,
        grid=(grid,),
        in_specs=[
            pl.BlockSpec((_BN, D), blk),
            pl.BlockSpec((NC, _BN, D), lambda i: (0, i, 0)),
            pl.BlockSpec((D, D), whole),
            pl.BlockSpec((D, D), whole),
            pl.BlockSpec((1, D), whole),
            pl.BlockSpec((D, D), whole),
            pl.BlockSpec((1, D), whole),
        ],
        out_specs=pl.BlockSpec((_BN, D), blk),
        out_shape=jax.ShapeDtypeStruct((N, D), jnp.float32),
    )(nodes, partials, w2a, w2b, b2.reshape(1, D), w3, b3.reshape(1, D))


def kernel(nodes, edge_index, edge_features, W1, b1, W2, b2, W3, b3):
    src = edge_index[0]
    tgt = edge_index[1]
    W1a, W1b = W1[:D], W1[D:]
    W2a, W2b = W2[:D], W2[D:]

    p, efp0 = _node_and_edge_proj(nodes, W1a, b1, edge_features, W1b)
    efp1 = _edge_proj(edge_features, W1b)
    zeros = jnp.zeros((NC, N, D), jnp.float32)
    part0 = _sc_half0(p, efp0, src, tgt, zeros)
    part1 = _sc_half1(p, efp1, src, tgt, part0)
    return _node_update(nodes, part1, W2a, W2b, b2, W3, b3)


# speedup vs baseline: 1.2407x; 1.2407x over previous
import functools

import jax
import jax.numpy as jnp
from jax import lax
from jax.experimental import pallas as pl
from jax.experimental.pallas import tpu as pltpu
from jax.experimental.pallas import tpu_sc as plsc

N, E, D = 10000, 320000, 128
L = 16
NC, NS = 2, 16
NW = NC * NS
EH = E // 2
EW = EH // NW
C = 40
CHUNKS = EW // C
PAIRS = CHUNKS // 2

_HI = lax.Precision.HIGHEST


def _dot(a, b):
    return lax.dot_general(a, b, (((1,), (0,)), ((), ())),
                           precision=_HI, preferred_element_type=jnp.float32)


def _silu(x):
    return x * jax.nn.sigmoid(x)


_BE = 1600


def _fused_proj_body(nodes_ref, w1a_ref, b1_ref, ef_ref, w1b_ref,
                     p_ref, efp_ref):
    @pl.when(pl.program_id(0) == 0)
    def _():
        p_ref[...] = _dot(nodes_ref[...], w1a_ref[...]) + b1_ref[...]

    efp_ref[...] = _dot(ef_ref[...], w1b_ref[...])


def _node_and_edge_proj(nodes, w1a, b1, ef, w1b):
    grid = EH // _BE + 1
    emap = lambda i: (jnp.maximum(i - 1, 0), 0)
    eomap = lambda i: (jnp.maximum(i - 1, 0), 0)
    whole = lambda i: (0, 0)
    return pl.pallas_call(
        _fused_proj_body,
        grid=(grid,),
        in_specs=[
            pl.BlockSpec((N, D), whole),
            pl.BlockSpec((D, D), whole),
            pl.BlockSpec((1, D), whole),
            pl.BlockSpec((_BE, D), emap),
            pl.BlockSpec((D, D), whole),
        ],
        out_specs=[
            pl.BlockSpec((N, D), whole),
            pl.BlockSpec((_BE, D), eomap),
        ],
        out_shape=[
            jax.ShapeDtypeStruct((N, D), jnp.float32),
            jax.ShapeDtypeStruct((EH, D), jnp.float32),
        ],
    )(nodes, w1a, b1.reshape(1, D), ef, w1b)


def _edge_proj_body(ef_ref, w_ref, out_ref):
    out_ref[...] = _dot(ef_ref[...], w_ref[...])


def _edge_proj(ef, w):
    grid = EH // _BE
    return pl.pallas_call(
        _edge_proj_body,
        grid=(grid,),
        in_specs=[
            pl.BlockSpec((_BE, D), lambda i: (i + EH // _BE, 0)),
            pl.BlockSpec((D, D), lambda i: (0, 0)),
        ],
        out_specs=pl.BlockSpec((_BE, D), lambda i: (i, 0)),
        out_shape=jax.ShapeDtypeStruct((EH, D), jnp.float32),
    )(ef, w)


_ZR = 640
_ZR_LAST = N - 15 * _ZR


def _sc_body(edge0, p_hbm, efp_hbm, src_hbm, tgt_hbm, init_hbm, out_hbm,
             srcall, tidx0, tidx1, g0, g1, e0, e1,
             acc, tsems, gsems, esems, ssems):
    c = lax.axis_index("c")
    s = lax.axis_index("s")
    wid = s * NC + c
    base_e = edge0 + wid * EW
    base_f = wid * EW
    tidx = (tidx0, tidx1)
    grows = (g0, g1)
    erows = (e0, e1)

    pltpu.async_copy(src_hbm.at[pl.ds(base_e, EW)], srcall, gsems.at[0])

    @pl.when(s < NS - 1)
    def _():
        pltpu.async_copy(init_hbm.at[c, pl.ds(s * _ZR, _ZR)],
                         acc.at[pl.ds(s * _ZR, _ZR)], tsems.at[0])
        pltpu.make_async_copy(init_hbm.at[c, pl.ds(s * _ZR, _ZR)],
                              acc.at[pl.ds(s * _ZR, _ZR)], tsems.at[0]).wait()

    @pl.when(s == NS - 1)
    def _():
        pltpu.async_copy(init_hbm.at[c, pl.ds(15 * _ZR, _ZR_LAST)],
                         acc.at[pl.ds(15 * _ZR, _ZR_LAST)], tsems.at[0])
        pltpu.make_async_copy(
            init_hbm.at[c, pl.ds(15 * _ZR, _ZR_LAST)],
            acc.at[pl.ds(15 * _ZR, _ZR_LAST)], tsems.at[0]).wait()

    pltpu.make_async_copy(src_hbm.at[pl.ds(base_e, EW)], srcall,
                          gsems.at[0]).wait()
    plsc.subcore_barrier()

    def fetch(b, t):
        pltpu.async_copy(tgt_hbm.at[pl.ds(base_e + t * C, C)], tidx[b],
                         tsems.at[b])
        pltpu.async_copy(p_hbm.at[srcall.at[pl.ds(t * C, C)]], grows[b],
                         gsems.at[b])
        pltpu.async_copy(efp_hbm.at[pl.ds(base_f + t * C, C)], erows[b],
                         esems.at[b])

    def wait_fetch(b, t):
        pltpu.make_async_copy(tgt_hbm.at[pl.ds(base_e + t * C, C)], tidx[b],
                              tsems.at[b]).wait()
        pltpu.make_async_copy(p_hbm.at[srcall.at[pl.ds(t * C, C)]], grows[b],
                              gsems.at[b]).wait()
        pltpu.make_async_copy(efp_hbm.at[pl.ds(base_f + t * C, C)], erows[b],
                              esems.at[b]).wait()

    def compute(b):
        gb = grows[b]
        eb = erows[b]

        def row_body(i):
            for j in range(D // L):
                x = gb[i, pl.ds(j * L, L)] + eb[i, pl.ds(j * L, L)]
                eb[i, pl.ds(j * L, L)] = x / (1.0 + jnp.exp(-x))

        plsc.parallel_loop(0, C, unroll=2)(row_body)

    def process(b, t):
        wait_fetch(b, t)
        compute(b)
        pltpu.async_copy(erows[b], acc.at[tidx[b]], ssems.at[b], add=True)
        pltpu.make_async_copy(erows[b], acc.at[tidx[b]], ssems.at[b]).wait()

        @pl.when(t + 2 < CHUNKS)
        def _():
            fetch(b, t + 2)

    fetch(0, 0)
    fetch(1, 1)

    def pair_body(it, carry):
        process(0, it * 2)
        process(1, it * 2 + 1)
        return carry

    lax.fori_loop(0, PAIRS, pair_body, 0)
    process(0, CHUNKS - 1)

    plsc.subcore_barrier()

    @pl.when(s < NS - 1)
    def _():
        pltpu.sync_copy(acc.at[pl.ds(s * _ZR, _ZR)],
                        out_hbm.at[c, pl.ds(s * _ZR, _ZR)])

    @pl.when(s == NS - 1)
    def _():
        pltpu.sync_copy(acc.at[pl.ds(15 * _ZR, _ZR_LAST)],
                        out_hbm.at[c, pl.ds(15 * _ZR, _ZR_LAST)])


def _make_sc_aggregate(edge0):
    @functools.partial(
        pl.kernel,
        out_type=jax.ShapeDtypeStruct((NC, N, D), jnp.float32),
        mesh=plsc.VectorSubcoreMesh(core_axis_name="c", subcore_axis_name="s"),
        scratch_types=[
            pltpu.VMEM((EW,), jnp.int32),
            pltpu.VMEM((C,), jnp.int32),
            pltpu.VMEM((C,), jnp.int32),
            pltpu.VMEM((C, D), jnp.float32),
            pltpu.VMEM((C, D), jnp.float32),
            pltpu.VMEM((C, D), jnp.float32),
            pltpu.VMEM((C, D), jnp.float32),
            pltpu.VMEM_SHARED((N, D), jnp.float32),
            pltpu.SemaphoreType.DMA((2,)),
            pltpu.SemaphoreType.DMA((2,)),
            pltpu.SemaphoreType.DMA((2,)),
            pltpu.SemaphoreType.DMA((2,)),
        ],
    )
    def sc_aggregate(*args):
        _sc_body(edge0, *args)

    return sc_aggregate


_sc_half0 = _make_sc_aggregate(0)
_sc_half1 = _make_sc_aggregate(EH)


_BN = 2000


def _update_body(nodes_ref, part_ref, w2a_ref, w2b_ref, b2_ref,
                 w3_ref, b3_ref, out_ref):
    nodes = nodes_ref[...]
    agg = part_ref[0] + part_ref[1]
    u = _silu(_dot(nodes, w2a_ref[...]) + _dot(agg, w2b_ref[...]) + b2_ref[...])
    out_ref[...] = nodes + _dot(u, w3_ref[...]) + b3_ref[...]


def _node_update(nodes, partials, w2a, w2b, b2, w3, b3):
    grid = N // _BN
    blk = lambda i: (i, 0)
    whole = lambda i: (0, 0)
    return pl.pallas_call(
        _update_body,
        grid=(grid,),
        in_specs=[
            pl.BlockSpec((_BN, D), blk),
            pl.BlockSpec((NC, _BN, D), lambda i: (0, i, 0)),
            pl.BlockSpec((D, D), whole),
            pl.BlockSpec((D, D), whole),
            pl.BlockSpec((1, D), whole),
            pl.BlockSpec((D, D), whole),
            pl.BlockSpec((1, D), whole),
        ],
        out_specs=pl.BlockSpec((_BN, D), blk),
        out_shape=jax.ShapeDtypeStruct((N, D), jnp.float32),
    )(nodes, partials, w2a, w2b, b2.reshape(1, D), w3, b3.reshape(1, D))


def kernel(nodes, edge_index, edge_features, W1, b1, W2, b2, W3, b3):
    src = edge_index[0]
    tgt = edge_index[1]
    W1a, W1b = W1[:D], W1[D:]
    W2a, W2b = W2[:D], W2[D:]

    p, efp0 = _node_and_edge_proj(nodes, W1a, b1, edge_features, W1b)
    efp1 = _edge_proj(edge_features, W1b)
    zeros = jnp.zeros((NC, N, D), jnp.float32)
    part0 = _sc_half0(p, efp0, src, tgt, zeros)
    part1 = _sc_half1(p, efp1, src, tgt, part0)
    return _node_update(nodes, part1, W2a, W2b, b2, W3, b3)
